# baseline (speedup 1.0000x reference)
import functools

import jax
import jax.numpy as jnp
from jax import lax
from jax.experimental import pallas as pl
from jax.experimental.pallas import tpu as pltpu
from jax.experimental.pallas import tpu_sc as plsc

N_FINE = 100000
N_COARSE = 25000
VAL_DIM = 128
FE = 9
NF = 128
KDIM = FE * VAL_DIM

NC_SC = 2
NS_SC = 16
NW = NC_SC * NS_SC

M_BLK = 512
M_PAD = 25088
CHUNK = 128
PKROWS = CHUNK // 2
LANE = 16


def _make_sc_gather(tot_rows):
    n_chunks = tot_rows // CHUNK
    base_iters = n_chunks // NW
    rem = n_chunks - base_iters * NW
    max_iters = base_iters + (1 if rem else 0)
    stage_rows = -(-(max_iters + 8) // 8) * 8
    chunks_pad = n_chunks + stage_rows
    mesh = plsc.VectorSubcoreMesh(core_axis_name="c", subcore_axis_name="s")

    @functools.partial(
        pl.kernel,
        mesh=mesh,
        out_type=jax.ShapeDtypeStruct((tot_rows // 2, VAL_DIM), jnp.int32),
        scratch_types=[
            pltpu.VMEM((stage_rows, CHUNK), jnp.int32),
            pltpu.VMEM((2, CHUNK, VAL_DIM), jnp.int32),
            pltpu.VMEM((2, PKROWS, VAL_DIM), jnp.int32),
            pltpu.SemaphoreType.DMA((2,)),
            pltpu.SemaphoreType.DMA((2,)),
        ],
    )
    def gather_kernel(fine_hbm, idx_hbm, out_hbm, idx_v, rows_v, pk_v,
                      gsem, wsem):
        wid = lax.axis_index("s") * NC_SC + lax.axis_index("c")
        first = wid * base_iters + jnp.minimum(wid, rem)
        n = base_iters + (wid < rem).astype(jnp.int32)

        aligned = pl.multiple_of((first // 8) * 8, 8)
        off = first - aligned
        pltpu.sync_copy(idx_hbm.at[pl.ds(aligned, stage_rows)], idx_v)

        def start_gather(j, slot):
            pltpu.async_copy(fine_hbm.at[idx_v.at[j + off]], rows_v.at[slot],
                             gsem.at[slot])

        def wait_gather(slot):
            pltpu.make_async_copy(fine_hbm.at[idx_v.at[0]], rows_v.at[slot],
                                  gsem.at[slot]).wait()

        def start_write(j, slot):
            dst = pl.multiple_of((first + j) * PKROWS, 8)
            pltpu.async_copy(pk_v.at[slot], out_hbm.at[pl.ds(dst, PKROWS)],
                             wsem.at[slot])

        def wait_write(slot):
            pltpu.make_async_copy(pk_v.at[slot],
                                  out_hbm.at[pl.ds(0, PKROWS)],
                                  wsem.at[slot]).wait()

        def bf16_hi_bits(xi):
            lsb = lax.shift_right_logical(xi, 16) & 1
            return xi + 32767 + lsb

        def convert(slot):
            def qbody(q, carry):
                for g in range(VAL_DIM // LANE):
                    sl = pl.ds(g * LANE, LANE)
                    ra = bf16_hi_bits(rows_v[slot, 2 * q, sl])
                    rb = bf16_hi_bits(rows_v[slot, 2 * q + 1, sl])
                    pk_v[slot, q, sl] = (lax.shift_right_logical(ra, 16)
                                         | (rb & jnp.int32(-65536)))
                return carry

            lax.fori_loop(0, PKROWS, qbody, 0)

        start_gather(0, 0)

        def body(j, carry):
            slot = lax.rem(j, 2)
            nslot = 1 - slot

            wait_gather(slot)

            @pl.when(j + 1 < n)
            def _():
                start_gather(j + 1, nslot)

            @pl.when(j >= 2)
            def _():
                wait_write(slot)

            convert(slot)
            start_write(j, slot)
            return carry

        lax.fori_loop(0, n, body, 0)

        @pl.when(n >= 2)
        def _():
            wait_write(lax.rem(n, 2))

        wait_write(lax.rem(n - 1, 2))

    return gather_kernel, chunks_pad


def _mm_body(a_ref, w_ref, o_ref):
    a = jnp.concatenate(
        [pltpu.bitcast(a_ref[i], jnp.bfloat16) for i in range(FE)], axis=1)
    o_ref[...] = jnp.dot(a, w_ref[...], preferred_element_type=jnp.float32)


def _tc_matmul(a3, w, m_pad, m_out):
    grid = (m_pad // M_BLK,)
    return pl.pallas_call(
        _mm_body,
        grid=grid,
        in_specs=[
            pl.BlockSpec((FE, M_BLK // 2, VAL_DIM), lambda m: (0, m, 0)),
            pl.BlockSpec((KDIM, NF), lambda m: (0, 0)),
        ],
        out_specs=pl.BlockSpec((M_BLK, NF), lambda m: (m, 0)),
        out_shape=jax.ShapeDtypeStruct((m_out, NF), jnp.float32),
    )(a3, w)


def kernel(lattice_fine_values, coarse_neighbor_indices, weight):
    idx = coarse_neighbor_indices.astype(jnp.int32)
    idx_t = jnp.pad(idx.T, ((0, 0), (0, M_PAD - N_COARSE)))
    w_bf = weight.astype(jnp.bfloat16)

    tot_rows = FE * M_PAD
    gather_fn, chunks_pad = _make_sc_gather(tot_rows)
    idx_flat = idx_t.reshape(-1)
    idx2d = jnp.pad(idx_flat, (0, chunks_pad * CHUNK - tot_rows))
    idx2d = idx2d.reshape(chunks_pad, CHUNK)
    fine_bits = jax.lax.bitcast_convert_type(lattice_fine_values, jnp.int32)
    packed = gather_fn(fine_bits, idx2d)
    a3 = packed.reshape(FE, M_PAD // 2, VAL_DIM)
    return _tc_matmul(a3, w_bf, M_PAD, N_COARSE)


# speedup vs baseline: 1.3896x; 1.3896x over previous
import functools

import jax
import jax.numpy as jnp
from jax import lax
from jax.experimental import pallas as pl
from jax.experimental.pallas import tpu as pltpu
from jax.experimental.pallas import tpu_sc as plsc

N_FINE = 100000
N_COARSE = 25000
VAL_DIM = 128
FE = 9
NF = 128
KDIM = FE * VAL_DIM

NC_SC = 2
NS_SC = 16
NW = NC_SC * NS_SC

M_BLK = 512
M_PAD = 25088
CHUNK = 128
PKROWS = CHUNK // 2
LANE = 16


def _make_sc_gather(tot_rows):
    n_chunks = tot_rows // CHUNK
    base_iters = n_chunks // NW
    rem = n_chunks - base_iters * NW
    max_iters = base_iters + (1 if rem else 0)
    stage_rows = -(-(max_iters + 8) // 8) * 8
    chunks_pad = n_chunks + stage_rows
    mesh = plsc.VectorSubcoreMesh(core_axis_name="c", subcore_axis_name="s")

    @functools.partial(
        pl.kernel,
        mesh=mesh,
        out_type=jax.ShapeDtypeStruct((tot_rows // 2, VAL_DIM), jnp.int32),
        scratch_types=[
            pltpu.VMEM((stage_rows, CHUNK), jnp.int32),
            pltpu.VMEM((2, CHUNK, VAL_DIM), jnp.int32),
            pltpu.VMEM((2, PKROWS, VAL_DIM), jnp.int32),
            pltpu.SemaphoreType.DMA((2,)),
            pltpu.SemaphoreType.DMA((2,)),
        ],
    )
    def gather_kernel(fine_f32_hbm, idx_hbm, out_hbm, idx_v, rows_v, pk_v,
                      gsem, wsem):
        fine_hbm = fine_f32_hbm.bitcast(jnp.int32)
        wid = lax.axis_index("s") * NC_SC + lax.axis_index("c")
        first = wid * base_iters + jnp.minimum(wid, rem)
        n = base_iters + (wid < rem).astype(jnp.int32)

        aligned = pl.multiple_of((first // 8) * 8, 8)
        off = first - aligned
        pltpu.sync_copy(idx_hbm.at[pl.ds(aligned, stage_rows)], idx_v)

        def start_gather(j, slot):
            pltpu.async_copy(fine_hbm.at[idx_v.at[j + off]], rows_v.at[slot],
                             gsem.at[slot])

        def wait_gather(slot):
            pltpu.make_async_copy(fine_hbm.at[idx_v.at[0]], rows_v.at[slot],
                                  gsem.at[slot]).wait()

        def start_write(j, slot):
            dst = pl.multiple_of((first + j) * PKROWS, 8)
            pltpu.async_copy(pk_v.at[slot], out_hbm.at[pl.ds(dst, PKROWS)],
                             wsem.at[slot])

        def wait_write(slot):
            pltpu.make_async_copy(pk_v.at[slot],
                                  out_hbm.at[pl.ds(0, PKROWS)],
                                  wsem.at[slot]).wait()

        def convert(slot):
            def qbody(q, carry):
                for g in range(VAL_DIM // LANE):
                    sl = pl.ds(g * LANE, LANE)
                    ra = rows_v[slot, 2 * q, sl]
                    rb = rows_v[slot, 2 * q + 1, sl]
                    pk_v[slot, q, sl] = (lax.shift_right_logical(ra, 16)
                                         | (rb & jnp.int32(-65536)))
                return carry

            lax.fori_loop(0, PKROWS, qbody, 0)

        start_gather(0, 0)

        def body(j, carry):
            slot = lax.rem(j, 2)
            nslot = 1 - slot

            wait_gather(slot)

            @pl.when(j + 1 < n)
            def _():
                start_gather(j + 1, nslot)

            @pl.when(j >= 2)
            def _():
                wait_write(slot)

            convert(slot)
            start_write(j, slot)
            return carry

        lax.fori_loop(0, n, body, 0)

        @pl.when(n >= 2)
        def _():
            wait_write(lax.rem(n, 2))

        wait_write(lax.rem(n - 1, 2))

    return gather_kernel, chunks_pad


def _mm_body(a_ref, w_ref, o_ref):
    a = jnp.concatenate(
        [pltpu.bitcast(a_ref[i], jnp.bfloat16) for i in range(FE)], axis=1)
    o_ref[...] = jnp.dot(a, w_ref[...], preferred_element_type=jnp.float32)


def _tc_matmul(a3, w, m_pad, m_out):
    grid = (m_pad // M_BLK,)
    return pl.pallas_call(
        _mm_body,
        grid=grid,
        in_specs=[
            pl.BlockSpec((FE, M_BLK // 2, VAL_DIM), lambda m: (0, m, 0)),
            pl.BlockSpec((KDIM, NF), lambda m: (0, 0)),
        ],
        out_specs=pl.BlockSpec((M_BLK, NF), lambda m: (m, 0)),
        out_shape=jax.ShapeDtypeStruct((m_out, NF), jnp.float32),
    )(a3, w)


def kernel(lattice_fine_values, coarse_neighbor_indices, weight):
    idx = coarse_neighbor_indices.astype(jnp.int32)
    idx_t = jnp.pad(idx.T, ((0, 0), (0, M_PAD - N_COARSE)))
    w_bf = weight.astype(jnp.bfloat16)

    tot_rows = FE * M_PAD
    gather_fn, chunks_pad = _make_sc_gather(tot_rows)
    idx_flat = idx_t.reshape(-1)
    idx2d = jnp.pad(idx_flat, (0, chunks_pad * CHUNK - tot_rows))
    idx2d = idx2d.reshape(chunks_pad, CHUNK)
    packed = gather_fn(lattice_fine_values, idx2d)
    a3 = packed.reshape(FE, M_PAD // 2, VAL_DIM)
    return _tc_matmul(a3, w_bf, M_PAD, N_COARSE)


# speedup vs baseline: 1.4954x; 1.0761x over previous
import functools

import jax
import jax.numpy as jnp
from jax import lax
from jax.experimental import pallas as pl
from jax.experimental.pallas import tpu as pltpu
from jax.experimental.pallas import tpu_sc as plsc

N_FINE = 100000
N_COARSE = 25000
VAL_DIM = 128
FE = 9
NF = 128
KDIM = FE * VAL_DIM

NC_SC = 2
NS_SC = 16
NW = NC_SC * NS_SC

M_BLK = 896
M_PAD = 25088
CHUNK = 128
PKROWS = CHUNK // 2
LANE = 16
NBUF = 4


def _make_sc_gather(tot_rows):
    n_chunks = tot_rows // CHUNK
    base_iters = n_chunks // NW
    rem = n_chunks - base_iters * NW
    max_iters = base_iters + (1 if rem else 0)
    stage_rows = -(-(max_iters + 8) // 8) * 8
    chunks_pad = n_chunks + stage_rows
    mesh = plsc.VectorSubcoreMesh(core_axis_name="c", subcore_axis_name="s")

    @functools.partial(
        pl.kernel,
        mesh=mesh,
        out_type=jax.ShapeDtypeStruct((tot_rows // 2, VAL_DIM), jnp.int32),
        scratch_types=[
            pltpu.VMEM((stage_rows, CHUNK), jnp.int32),
            pltpu.VMEM((NBUF, CHUNK, VAL_DIM), jnp.int32),
            pltpu.VMEM((2, PKROWS, VAL_DIM), jnp.int32),
            pltpu.SemaphoreType.DMA((NBUF,)),
            pltpu.SemaphoreType.DMA((2,)),
        ],
    )
    def gather_kernel(fine_f32_hbm, idx_hbm, out_hbm, idx_v, rows_v, pk_v,
                      gsem, wsem):
        fine_hbm = fine_f32_hbm.bitcast(jnp.int32)
        wid = lax.axis_index("s") * NC_SC + lax.axis_index("c")
        first = wid * base_iters + jnp.minimum(wid, rem)
        n = base_iters + (wid < rem).astype(jnp.int32)

        aligned = pl.multiple_of((first // 8) * 8, 8)
        off = first - aligned
        pltpu.sync_copy(idx_hbm.at[pl.ds(aligned, stage_rows)], idx_v)

        def start_gather(j, slot):
            pltpu.async_copy(fine_hbm.at[idx_v.at[j + off]], rows_v.at[slot],
                             gsem.at[slot])

        def wait_gather(slot):
            pltpu.make_async_copy(fine_hbm.at[idx_v.at[0]], rows_v.at[slot],
                                  gsem.at[slot]).wait()

        def start_write(j, pslot):
            dst = pl.multiple_of((first + j) * PKROWS, 8)
            pltpu.async_copy(pk_v.at[pslot], out_hbm.at[pl.ds(dst, PKROWS)],
                             wsem.at[pslot])

        def wait_write(slot):
            pltpu.make_async_copy(pk_v.at[slot],
                                  out_hbm.at[pl.ds(0, PKROWS)],
                                  wsem.at[slot]).wait()

        def convert(slot, pslot):
            def qbody(q, carry):
                for g in range(VAL_DIM // LANE):
                    sl = pl.ds(g * LANE, LANE)
                    ra = rows_v[slot, 2 * q, sl]
                    rb = rows_v[slot, 2 * q + 1, sl]
                    pk_v[pslot, q, sl] = (lax.shift_right_logical(ra, 16)
                                          | (rb & jnp.int32(-65536)))
                return carry

            lax.fori_loop(0, PKROWS, qbody, 0)

        for b in range(NBUF - 1):
            @pl.when(b < n)
            def _(b=b):
                start_gather(b, b)

        def body(j, carry):
            slot = lax.rem(j, NBUF)
            pslot = lax.rem(j, 2)

            wait_gather(slot)

            @pl.when(j + NBUF - 1 < n)
            def _():
                start_gather(j + NBUF - 1, lax.rem(j + NBUF - 1, NBUF))

            @pl.when(j >= 2)
            def _():
                wait_write(pslot)

            convert(slot, pslot)
            start_write(j, pslot)
            return carry

        lax.fori_loop(0, n, body, 0)

        @pl.when(n >= 2)
        def _():
            wait_write(lax.rem(n, 2))

        wait_write(lax.rem(n - 1, 2))

    return gather_kernel, chunks_pad


def _mm_body(a_ref, w_ref, o_ref):
    a = jnp.concatenate(
        [pltpu.bitcast(a_ref[i], jnp.bfloat16) for i in range(FE)], axis=1)
    o_ref[...] = jnp.dot(a, w_ref[...], preferred_element_type=jnp.float32)


def _tc_matmul(a3, w, m_pad, m_out):
    grid = (m_pad // M_BLK,)
    return pl.pallas_call(
        _mm_body,
        grid=grid,
        in_specs=[
            pl.BlockSpec((FE, M_BLK // 2, VAL_DIM), lambda m: (0, m, 0)),
            pl.BlockSpec((KDIM, NF), lambda m: (0, 0)),
        ],
        out_specs=pl.BlockSpec((M_BLK, NF), lambda m: (m, 0)),
        out_shape=jax.ShapeDtypeStruct((m_out, NF), jnp.float32),
    )(a3, w)


def kernel(lattice_fine_values, coarse_neighbor_indices, weight):
    idx = coarse_neighbor_indices.astype(jnp.int32)
    idx_t = jnp.pad(idx.T, ((0, 0), (0, M_PAD - N_COARSE)))
    w_bf = weight.astype(jnp.bfloat16)

    tot_rows = FE * M_PAD
    gather_fn, chunks_pad = _make_sc_gather(tot_rows)
    idx_flat = idx_t.reshape(-1)
    idx2d = jnp.pad(idx_flat, (0, chunks_pad * CHUNK - tot_rows))
    idx2d = idx2d.reshape(chunks_pad, CHUNK)
    packed = gather_fn(lattice_fine_values, idx2d)
    a3 = packed.reshape(FE, M_PAD // 2, VAL_DIM)
    return _tc_matmul(a3, w_bf, M_PAD, N_COARSE)


# speedup vs baseline: 1.5005x; 1.0034x over previous
import functools

import jax
import jax.numpy as jnp
from jax import lax
from jax.experimental import pallas as pl
from jax.experimental.pallas import tpu as pltpu
from jax.experimental.pallas import tpu_sc as plsc

N_FINE = 100000
N_COARSE = 25000
VAL_DIM = 128
FE = 9
NF = 128
KDIM = FE * VAL_DIM

NC_SC = 2
NS_SC = 16
NW = NC_SC * NS_SC

M_BLK = 896
M_PAD = 25088
CHUNK = 128
PKROWS = CHUNK // 2
LANE = 16
NBUF = 4


def _make_sc_gather(tot_rows):
    n_chunks = tot_rows // CHUNK
    base_iters = n_chunks // NW
    rem = n_chunks - base_iters * NW
    max_iters = base_iters + (1 if rem else 0)
    stage_rows = -(-(max_iters + 8) // 8) * 8
    chunks_pad = n_chunks + stage_rows
    mesh = plsc.VectorSubcoreMesh(core_axis_name="c", subcore_axis_name="s")

    @functools.partial(
        pl.kernel,
        mesh=mesh,
        out_type=jax.ShapeDtypeStruct((tot_rows // 2, VAL_DIM), jnp.int32),
        scratch_types=[
            pltpu.VMEM((stage_rows, CHUNK), jnp.int32),
            pltpu.VMEM((NBUF, CHUNK, VAL_DIM), jnp.int32),
            pltpu.VMEM((2, PKROWS, VAL_DIM), jnp.int32),
            pltpu.SemaphoreType.DMA((NBUF,)),
            pltpu.SemaphoreType.DMA((2,)),
        ],
    )
    def gather_kernel(fine_f32_hbm, idx_hbm, out_hbm, idx_v, rows_v, pk_v,
                      gsem, wsem):
        fine_hbm = fine_f32_hbm.bitcast(jnp.int32)
        wid = lax.axis_index("s") * NC_SC + lax.axis_index("c")
        first = wid * base_iters + jnp.minimum(wid, rem)
        n = base_iters + (wid < rem).astype(jnp.int32)

        aligned = pl.multiple_of((first // 8) * 8, 8)
        off = first - aligned
        pltpu.sync_copy(idx_hbm.at[pl.ds(aligned, stage_rows)], idx_v)

        def start_gather(j, slot):
            pltpu.async_copy(fine_hbm.at[idx_v.at[j + off]], rows_v.at[slot],
                             gsem.at[slot])

        def wait_gather(slot):
            pltpu.make_async_copy(fine_hbm.at[idx_v.at[0]], rows_v.at[slot],
                                  gsem.at[slot]).wait()

        def start_write(j, pslot):
            dst = pl.multiple_of((first + j) * PKROWS, 8)
            pltpu.async_copy(pk_v.at[pslot], out_hbm.at[pl.ds(dst, PKROWS)],
                             wsem.at[pslot])

        def wait_write(slot):
            pltpu.make_async_copy(pk_v.at[slot],
                                  out_hbm.at[pl.ds(0, PKROWS)],
                                  wsem.at[slot]).wait()

        def convert(slot, pslot):
            def qbody(q8, carry):
                for dq in range(8):
                    q = q8 * 8 + dq
                    for g in range(VAL_DIM // LANE):
                        sl = pl.ds(g * LANE, LANE)
                        ra = rows_v[slot, 2 * q, sl]
                        rb = rows_v[slot, 2 * q + 1, sl]
                        pk_v[pslot, q, sl] = (lax.shift_right_logical(ra, 16)
                                              | (rb & jnp.int32(-65536)))
                return carry

            lax.fori_loop(0, PKROWS // 8, qbody, 0)

        for b in range(NBUF - 1):
            @pl.when(b < n)
            def _(b=b):
                start_gather(b, b)

        def body(j, carry):
            slot = lax.rem(j, NBUF)
            pslot = lax.rem(j, 2)

            wait_gather(slot)

            @pl.when(j + NBUF - 1 < n)
            def _():
                start_gather(j + NBUF - 1, lax.rem(j + NBUF - 1, NBUF))

            @pl.when(j >= 2)
            def _():
                wait_write(pslot)

            convert(slot, pslot)
            start_write(j, pslot)
            return carry

        lax.fori_loop(0, n, body, 0)

        @pl.when(n >= 2)
        def _():
            wait_write(lax.rem(n, 2))

        wait_write(lax.rem(n - 1, 2))

    return gather_kernel, chunks_pad


def _mm_body(a_ref, w_ref, o_ref):
    a = jnp.concatenate(
        [pltpu.bitcast(a_ref[i], jnp.bfloat16) for i in range(FE)], axis=1)
    o_ref[...] = jnp.dot(a, w_ref[...], preferred_element_type=jnp.float32)


def _tc_matmul(a3, w, m_pad, m_out):
    grid = (m_pad // M_BLK,)
    return pl.pallas_call(
        _mm_body,
        grid=grid,
        in_specs=[
            pl.BlockSpec((FE, M_BLK // 2, VAL_DIM), lambda m: (0, m, 0)),
            pl.BlockSpec((KDIM, NF), lambda m: (0, 0)),
        ],
        out_specs=pl.BlockSpec((M_BLK, NF), lambda m: (m, 0)),
        out_shape=jax.ShapeDtypeStruct((m_out, NF), jnp.float32),
    )(a3, w)


def kernel(lattice_fine_values, coarse_neighbor_indices, weight):
    idx = coarse_neighbor_indices.astype(jnp.int32)
    idx_t = jnp.pad(idx.T, ((0, 0), (0, M_PAD - N_COARSE)))
    w_bf = weight.astype(jnp.bfloat16)

    tot_rows = FE * M_PAD
    gather_fn, chunks_pad = _make_sc_gather(tot_rows)
    idx_flat = idx_t.reshape(-1)
    idx2d = jnp.pad(idx_flat, (0, chunks_pad * CHUNK - tot_rows))
    idx2d = idx2d.reshape(chunks_pad, CHUNK)
    packed = gather_fn(lattice_fine_values, idx2d)
    a3 = packed.reshape(FE, M_PAD // 2, VAL_DIM)
    return _tc_matmul(a3, w_bf, M_PAD, N_COARSE)
